# baseline (speedup 1.0000x reference)
import numpy as np
import jax
import jax.numpy as jnp
from jax.experimental import pallas as pl
from jax.experimental.pallas import tpu as pltpu
from jax.experimental.pallas import tpu_sc as plsc

B, S, D, H, L, T, INTER, NCLS = 8, 512, 128, 8, 4, 64, 512, 2
VOCAB = 100000
DH = D // H
SCALE = float(1.0 / np.sqrt(DH))
NEG = -1e9


def _sinusoidal_pos(S, D):
    pos = np.arange(S)[:, None].astype(np.float64)
    i = np.arange(D)[None, :]
    angle = pos / np.power(10000.0, (2 * (i // 2)) / D)
    pe = np.where(i % 2 == 0, np.sin(angle), np.cos(angle))
    return pe.astype(np.float32)


_PE = _sinusoidal_pos(S, D)


_GATHER_WINDOW = 128


def _sc_gather(emb_table, flat_ids):
    n = flat_ids.shape[1]
    mesh = plsc.VectorSubcoreMesh(core_axis_name="c", subcore_axis_name="s")

    @pl.kernel(
        out_type=jax.ShapeDtypeStruct((n, emb_table.shape[1]), emb_table.dtype),
        mesh=mesh,
    )
    def emb_gather(tbl_hbm, ids_hbm, out_hbm):
        def body(ids_vmem, out_vmem):
            pltpu.sync_copy(tbl_hbm.at[ids_vmem.at[0]], out_vmem)

        pltpu.emit_pipeline(
            body,
            grid=(n // _GATHER_WINDOW,),
            in_specs=[
                pl.BlockSpec((1, _GATHER_WINDOW), index_map=lambda i: (0, i))
            ],
            out_specs=[
                pl.BlockSpec(
                    (_GATHER_WINDOW, emb_table.shape[1]),
                    index_map=lambda i: (i, 0),
                )
            ],
            core_axis_name=("c", "s"),
            dimension_semantics=(pltpu.PARALLEL,),
        )(ids_hbm, out_hbm)

    return emb_gather(emb_table, flat_ids)


def _dot(a, b):
    return jnp.dot(a, b, preferred_element_type=jnp.float32)


def _gat_kernel(
    h0_ref, pe_ref, adj_ref, et_ref, etab_ref,
    wq_ref, wk_ref, wv_ref, wo_ref, w1_ref, w2_ref,
    b1_ref, b2_ref, wcls_ref, bcls_ref,
    out_ref,
    w_s,
):
    b = pl.program_id(0)

    @pl.when(b == 0)
    def _():
        etab = etab_ref[...]
        for li in range(L):
            wq = wq_ref[li]
            wk = wk_ref[li]
            wv = wv_ref[li]
            wo = wo_ref[li]
            for h in range(H):
                sl = slice(h * DH, (h + 1) * DH)
                wq_h = wq[:, sl] * SCALE
                a_h = jax.lax.dot_general(
                    wq_h, wk[:, sl], (((1,), (1,)), ((), ())),
                    preferred_element_type=jnp.float32)
                b_h = jax.lax.dot_general(
                    wq_h, etab[:, sl], (((1,), (1,)), ((), ())),
                    preferred_element_type=jnp.float32)
                c_h = _dot(wv[:, sl], wo[sl, :])
                w_s[li, h] = jnp.concatenate(
                    [a_h, c_h, b_h, jnp.zeros((D, D - T), jnp.float32)],
                    axis=1).astype(jnp.bfloat16)

    row = jax.lax.broadcasted_iota(jnp.int32, (S, S), 0)
    col = jax.lax.broadcasted_iota(jnp.int32, (S, S), 1)
    mask = (adj_ref[0].astype(jnp.int32) > 0) | (row == col)
    gidx = jnp.where(mask, et_ref[0].astype(jnp.int32), T)
    negpad = jnp.concatenate(
        [jnp.zeros((S, T), jnp.float32), jnp.full((S, D - T), NEG, jnp.float32)],
        axis=1)

    hb = h0_ref[0] + pe_ref[...]
    for l in range(L):
        hb16 = hb.astype(jnp.bfloat16)
        acc = jnp.zeros((S, D), jnp.float32)
        for h in range(H):
            pcv = _dot(hb16, w_s[l, h])
            p = pcv[:, :D].astype(jnp.bfloat16)
            qk = jax.lax.dot_general(
                p, hb16, (((1,), (1,)), ((), ())),
                preferred_element_type=jnp.float32)
            tab = pcv[:, 2 * D:] + negpad
            t = qk + jnp.take_along_axis(tab, gidx, axis=1)
            e = jnp.exp(t.astype(jnp.bfloat16))
            v = pcv[:, D:2 * D].astype(jnp.bfloat16)
            v2 = jnp.concatenate([v, jnp.ones((S, D), jnp.bfloat16)], axis=1)
            uv = _dot(e, v2)
            acc = acc + uv[:, :D] / uv[:, D:]

        h1 = hb + acc
        f = jnp.maximum(
            _dot(h1.astype(jnp.bfloat16), w1_ref[l].astype(jnp.bfloat16))
            + b1_ref[l], 0.0)
        hb = h1 + _dot(f, w2_ref[l]) + b2_ref[l]

    cls = hb[0:1, :]
    out_ref[pl.ds(b, 1), :] = _dot(cls, wcls_ref[...]) + bcls_ref[...]


def _tc_forward(h0, adj8, et8, edge_table, Wq, Wk, Wv, Wo, W1, W2,
                b1r, b2r, W_cls, bclsr):
    bl = h0.shape[0]
    grid = (bl,)
    return pl.pallas_call(
        _gat_kernel,
        grid=grid,
        in_specs=[
            pl.BlockSpec((1, S, D), lambda b: (b, 0, 0)),
            pl.BlockSpec((S, D), lambda b: (0, 0)),
            pl.BlockSpec((1, S, S), lambda b: (b, 0, 0)),
            pl.BlockSpec((1, S, S), lambda b: (b, 0, 0)),
            pl.BlockSpec((T, D), lambda b: (0, 0)),
            pl.BlockSpec((L, D, D), lambda b: (0, 0, 0)),
            pl.BlockSpec((L, D, D), lambda b: (0, 0, 0)),
            pl.BlockSpec((L, D, D), lambda b: (0, 0, 0)),
            pl.BlockSpec((L, D, D), lambda b: (0, 0, 0)),
            pl.BlockSpec((L, D, INTER), lambda b: (0, 0, 0)),
            pl.BlockSpec((L, INTER, D), lambda b: (0, 0, 0)),
            pl.BlockSpec((L, 1, INTER), lambda b: (0, 0, 0)),
            pl.BlockSpec((L, 1, D), lambda b: (0, 0, 0)),
            pl.BlockSpec((D, NCLS), lambda b: (0, 0)),
            pl.BlockSpec((1, NCLS), lambda b: (0, 0)),
        ],
        out_specs=pl.BlockSpec((bl, NCLS), lambda b: (0, 0)),
        out_shape=jax.ShapeDtypeStruct((bl, NCLS), jnp.float32),
        scratch_shapes=[
            pltpu.VMEM((L, H, D, 3 * D), jnp.bfloat16),
        ],
        compiler_params=pltpu.CompilerParams(
            dimension_semantics=("arbitrary",),
        ),
    )(h0, jnp.asarray(_PE), adj8, et8, edge_table, Wq, Wk, Wv, Wo, W1, W2,
      b1r, b2r, W_cls, bclsr)


def _sharded_forward(flat_ids, tbl_shard, adj8, et8, edge_table,
                     Wq, Wk, Wv, Wo, W1, W2, b1r, b2r, W_cls, bclsr):
    part = jax.lax.axis_index("x")
    half = VOCAB // 2
    off = flat_ids - part * half
    ids_loc = jnp.clip(off, 0, half - 1)
    g = _sc_gather(tbl_shard, ids_loc)
    valid = (off >= 0) & (off < half)
    g = jnp.where(valid.reshape(B * S, 1), g, 0.0)
    h0 = jax.lax.psum(g, "x").reshape(B, S, D)
    h0_l = jax.lax.dynamic_slice_in_dim(h0, part * (B // 2), B // 2, axis=0)
    return _tc_forward(h0_l, adj8, et8, edge_table, Wq, Wk, Wv, Wo,
                       W1, W2, b1r, b2r, W_cls, bclsr)


def kernel(word_ids, adj, edge_types, emb_table, edge_table,
           Wq, Wk, Wv, Wo, W1, W2, b1, b2, W_cls, b_cls):
    flat_ids = word_ids.astype(jnp.int32).reshape(1, B * S)
    adj8 = adj.astype(jnp.int8)
    et8 = edge_types.astype(jnp.int8)
    b1r = b1.reshape(L, 1, INTER)
    b2r = b2.reshape(L, 1, D)
    bclsr = b_cls.reshape(1, NCLS)

    devs = jax.devices()
    if len(devs) >= 2:
        from jax.sharding import Mesh, PartitionSpec as P
        mesh = Mesh(np.array(devs[:2]), ("x",))
        specs = (P(), P("x"), P("x"), P("x")) + (P(),) * 11
        fwd = jax.shard_map(_sharded_forward, mesh=mesh,
                            in_specs=specs, out_specs=P("x"),
                            check_vma=False)
        return fwd(flat_ids, emb_table, adj8, et8, edge_table,
                   Wq, Wk, Wv, Wo, W1, W2, b1r, b2r, W_cls, bclsr)

    h0 = _sc_gather(emb_table, flat_ids).reshape(B, S, D)
    return _tc_forward(h0, adj8, et8, edge_table, Wq, Wk, Wv, Wo,
                       W1, W2, b1r, b2r, W_cls, bclsr)


# speedup vs baseline: 2.7105x; 2.7105x over previous
import numpy as np
import jax
import jax.numpy as jnp
from jax.experimental import pallas as pl
from jax.experimental.pallas import tpu as pltpu
from jax.experimental.pallas import tpu_sc as plsc

B, S, D, H, L, T, INTER, NCLS = 8, 512, 128, 8, 4, 64, 512, 2
VOCAB = 100000
DH = D // H
SCALE = float(1.0 / np.sqrt(DH))
NEG = -1e9


def _sinusoidal_pos(S, D):
    pos = np.arange(S)[:, None].astype(np.float64)
    i = np.arange(D)[None, :]
    angle = pos / np.power(10000.0, (2 * (i // 2)) / D)
    pe = np.where(i % 2 == 0, np.sin(angle), np.cos(angle))
    return pe.astype(np.float32)


_PE = _sinusoidal_pos(S, D)


_GATHER_WINDOW = 128


def _sc_gather(emb_table, flat_ids):
    n = flat_ids.shape[1]
    mesh = plsc.VectorSubcoreMesh(core_axis_name="c", subcore_axis_name="s")

    @pl.kernel(
        out_type=jax.ShapeDtypeStruct((n, emb_table.shape[1]), emb_table.dtype),
        mesh=mesh,
    )
    def emb_gather(tbl_hbm, ids_hbm, out_hbm):
        def body(ids_vmem, out_vmem):
            pltpu.sync_copy(tbl_hbm.at[ids_vmem.at[0]], out_vmem)

        pltpu.emit_pipeline(
            body,
            grid=(n // _GATHER_WINDOW,),
            in_specs=[
                pl.BlockSpec((1, _GATHER_WINDOW), index_map=lambda i: (0, i))
            ],
            out_specs=[
                pl.BlockSpec(
                    (_GATHER_WINDOW, emb_table.shape[1]),
                    index_map=lambda i: (i, 0),
                )
            ],
            core_axis_name=("c", "s"),
            dimension_semantics=(pltpu.PARALLEL,),
        )(ids_hbm, out_hbm)

    return emb_gather(emb_table, flat_ids)


def _dot(a, b):
    return jnp.dot(a, b, preferred_element_type=jnp.float32)


def _gat_kernel(
    h0_ref, pe_ref, adj_ref, et_ref, etab_ref,
    wq_ref, wk_ref, wv_ref, wo_ref, w1_ref, w2_ref,
    b1_ref, b2_ref, wcls_ref, bcls_ref,
    out_ref,
    w_s,
):
    b = pl.program_id(0)

    @pl.when(b == 0)
    def _():
        etab = etab_ref[...]
        for li in range(L):
            wq = wq_ref[li]
            wk = wk_ref[li]
            wv = wv_ref[li]
            wo = wo_ref[li]
            for h in range(H):
                sl = slice(h * DH, (h + 1) * DH)
                wq_h = wq[:, sl] * SCALE
                a_h = jax.lax.dot_general(
                    wq_h, wk[:, sl], (((1,), (1,)), ((), ())),
                    preferred_element_type=jnp.float32)
                b_h = jax.lax.dot_general(
                    wq_h, etab[:, sl], (((1,), (1,)), ((), ())),
                    preferred_element_type=jnp.float32)
                c_h = _dot(wv[:, sl], wo[sl, :])
                w_s[li, h] = jnp.concatenate(
                    [a_h, c_h, b_h, jnp.zeros((D, D - T), jnp.float32)],
                    axis=1).astype(jnp.bfloat16)

    row = jax.lax.broadcasted_iota(jnp.int32, (S, S), 0)
    col = jax.lax.broadcasted_iota(jnp.int32, (S, S), 1)
    mask = (adj_ref[0].astype(jnp.int32) > 0) | (row == col)
    gidx = jnp.where(mask, et_ref[0].astype(jnp.int32), T)
    negpad = jnp.concatenate(
        [jnp.zeros((S, T), jnp.float32), jnp.full((S, D - T), NEG, jnp.float32)],
        axis=1)

    hb = h0_ref[0] + pe_ref[...]
    for l in range(L):
        hb16 = hb.astype(jnp.bfloat16)
        acc = jnp.zeros((S, D), jnp.float32)
        for h in range(H):
            pcv = _dot(hb16, w_s[l, h])
            p = pcv[:, :D].astype(jnp.bfloat16)
            qk = jax.lax.dot_general(
                p, hb16, (((1,), (1,)), ((), ())),
                preferred_element_type=jnp.float32)
            tab = pcv[:, 2 * D:] + negpad
            t = qk + jnp.take_along_axis(tab, gidx, axis=1)
            e = jnp.exp(t.astype(jnp.bfloat16))
            v = pcv[:, D:2 * D].astype(jnp.bfloat16)
            v2 = jnp.concatenate([v, jnp.ones((S, D), jnp.bfloat16)], axis=1)
            uv = _dot(e, v2)
            acc = acc + uv[:, :D] / uv[:, D:]

        h1 = hb + acc
        f = jnp.maximum(
            _dot(h1.astype(jnp.bfloat16), w1_ref[l].astype(jnp.bfloat16))
            + b1_ref[l], 0.0)
        hb = h1 + _dot(f, w2_ref[l]) + b2_ref[l]

    cls = hb[0:1, :]
    out_ref[pl.ds(b, 1), :] = _dot(cls, wcls_ref[...]) + bcls_ref[...]


def _tc_forward(h0, adj8, et8, edge_table, Wq, Wk, Wv, Wo, W1, W2,
                b1r, b2r, W_cls, bclsr):
    bl = h0.shape[0]
    grid = (bl,)
    return pl.pallas_call(
        _gat_kernel,
        grid=grid,
        in_specs=[
            pl.BlockSpec((1, S, D), lambda b: (b, 0, 0)),
            pl.BlockSpec((S, D), lambda b: (0, 0)),
            pl.BlockSpec((1, S, S), lambda b: (b, 0, 0)),
            pl.BlockSpec((1, S, S), lambda b: (b, 0, 0)),
            pl.BlockSpec((T, D), lambda b: (0, 0)),
            pl.BlockSpec((L, D, D), lambda b: (0, 0, 0)),
            pl.BlockSpec((L, D, D), lambda b: (0, 0, 0)),
            pl.BlockSpec((L, D, D), lambda b: (0, 0, 0)),
            pl.BlockSpec((L, D, D), lambda b: (0, 0, 0)),
            pl.BlockSpec((L, D, INTER), lambda b: (0, 0, 0)),
            pl.BlockSpec((L, INTER, D), lambda b: (0, 0, 0)),
            pl.BlockSpec((L, 1, INTER), lambda b: (0, 0, 0)),
            pl.BlockSpec((L, 1, D), lambda b: (0, 0, 0)),
            pl.BlockSpec((D, NCLS), lambda b: (0, 0)),
            pl.BlockSpec((1, NCLS), lambda b: (0, 0)),
        ],
        out_specs=pl.BlockSpec((bl, NCLS), lambda b: (0, 0)),
        out_shape=jax.ShapeDtypeStruct((bl, NCLS), jnp.float32),
        scratch_shapes=[
            pltpu.VMEM((L, H, D, 3 * D), jnp.bfloat16),
        ],
        compiler_params=pltpu.CompilerParams(
            dimension_semantics=("arbitrary",),
        ),
    )(h0, jnp.asarray(_PE), adj8, et8, edge_table, Wq, Wk, Wv, Wo, W1, W2,
      b1r, b2r, W_cls, bclsr)


def kernel(word_ids, adj, edge_types, emb_table, edge_table,
           Wq, Wk, Wv, Wo, W1, W2, b1, b2, W_cls, b_cls):
    flat_ids = word_ids.astype(jnp.int32).reshape(1, B * S)
    adj8 = adj.astype(jnp.int8)
    et8 = edge_types.astype(jnp.int8)
    b1r = b1.reshape(L, 1, INTER)
    b2r = b2.reshape(L, 1, D)
    bclsr = b_cls.reshape(1, NCLS)

    h0 = _sc_gather(emb_table, flat_ids).reshape(B, S, D)
    return _tc_forward(h0, adj8, et8, edge_table, Wq, Wk, Wv, Wo,
                       W1, W2, b1r, b2r, W_cls, bclsr)
